# Initial kernel scaffold; baseline (speedup 1.0000x reference)
import jax
import jax.numpy as jnp
from jax.experimental import pallas as pl


def kernel(x, edge_index, W_in, b_in, Wl, bl, W_out, b_out):
    raise NotImplementedError("write your pallas kernel here")


# speedup vs baseline: 6.7811x; 6.7811x over previous
import functools
import math

import jax
import jax.numpy as jnp
from jax import lax
from jax.experimental import pallas as pl
from jax.experimental.pallas import tpu as pltpu
from jax.experimental.pallas import tpu_sc as plsc

N = 10000
E = 320000
D = 128
H = 128
C = 40
L = 8
ALPHA = 0.1
LAMDA = 0.5

NC = 2
NS = 16
NT = NC * NS
CH = 128
NROW = 640
N_PAD = NS * NROW
E_PW = ((E // NT + CH - 1) // CH) * CH
N_CHUNK = E_PW // CH
E_PAD = E_PW * NT

_mesh = plsc.VectorSubcoreMesh(core_axis_name="c", subcore_axis_name="s")


def _tile_id():
    return lax.axis_index("c") * NS + lax.axis_index("s")


@functools.partial(
    pl.kernel,
    out_type=jax.ShapeDtypeStruct((NC, N_PAD, 16), jnp.float32),
    mesh=_mesh,
    scratch_types=[
        pltpu.VMEM((CH,), jnp.int32),
        pltpu.VMEM((CH, 16), jnp.float32),
        pltpu.VMEM((CH, 16), jnp.float32),
        pltpu.VMEM_SHARED((N_PAD, 16), jnp.float32),
    ],
)
def _hist_kernel(src_hbm, hist_out, idx_v, obuf, zbuf, hist_sh):
    c = lax.axis_index("c")
    s = lax.axis_index("s")
    t = c * NS + s

    def fill(i, _):
        zbuf[i, :] = jnp.zeros((16,), jnp.float32)
        obuf[i, :] = jnp.ones((16,), jnp.float32)
        return 0

    lax.fori_loop(0, CH, fill, 0)
    for r in range(NROW // CH):
        pltpu.sync_copy(zbuf, hist_sh.at[pl.ds(s * NROW + r * CH, CH)])
    plsc.subcore_barrier()

    def chunk(k, _):
        base = t * E_PW + k * CH
        pltpu.sync_copy(src_hbm.at[pl.ds(base, CH)], idx_v)
        pltpu.sync_copy(obuf, hist_sh.at[idx_v], add=True)
        return 0

    lax.fori_loop(0, N_CHUNK, chunk, 0)
    plsc.subcore_barrier()
    pltpu.sync_copy(hist_sh.at[pl.ds(s * NROW, NROW)],
                    hist_out.at[c, pl.ds(s * NROW, NROW)])


@functools.partial(
    pl.kernel,
    out_type=jax.ShapeDtypeStruct((NC, N_PAD, H), jnp.float32),
    mesh=_mesh,
    scratch_types=[
        pltpu.VMEM((CH,), jnp.int32),
        pltpu.VMEM((CH,), jnp.int32),
        pltpu.VMEM((CH, H), jnp.float32),
        pltpu.VMEM((CH, H), jnp.float32),
        pltpu.VMEM_SHARED((N_PAD, H), jnp.float32),
        pltpu.SemaphoreType.DMA,
    ],
)
def _spmm_kernel(g_hbm, src_hbm, dst_hbm, acc_out,
                 idx_s, idx_d, rows, zbuf, acc_sh, sem):
    c = lax.axis_index("c")
    s = lax.axis_index("s")
    t = c * NS + s

    def fill(i, _):
        for j in range(H // 16):
            zbuf[i, pl.ds(j * 16, 16)] = jnp.zeros((16,), jnp.float32)
        return 0

    lax.fori_loop(0, CH, fill, 0)
    for r in range(NROW // CH):
        pltpu.sync_copy(zbuf, acc_sh.at[pl.ds(s * NROW + r * CH, CH)])
    plsc.subcore_barrier()

    def chunk(k, _):
        base = t * E_PW + k * CH
        pltpu.sync_copy(src_hbm.at[pl.ds(base, CH)], idx_s)
        pltpu.sync_copy(dst_hbm.at[pl.ds(base, CH)], idx_d)
        pltpu.async_copy(g_hbm.at[idx_s], rows, sem).wait()
        pltpu.sync_copy(rows, acc_sh.at[idx_d], add=True)
        return 0

    lax.fori_loop(0, N_CHUNK, chunk, 0)
    plsc.subcore_barrier()
    pltpu.sync_copy(acc_sh.at[pl.ds(s * NROW, NROW)],
                    acc_out.at[c, pl.ds(s * NROW, NROW)])


_BLK = 2000
_GRID = N // _BLK


def _dinv_of(hist_ref):
    deg = hist_ref[0, :, 0:1] + hist_ref[1, :, 0:1] + 1.0
    return lax.rsqrt(deg)


def _pre_body(x_ref, w_ref, b_ref, hist_ref, h0_ref, g_ref):
    h = jnp.maximum(
        jnp.dot(x_ref[...], w_ref[...], preferred_element_type=jnp.float32)
        + b_ref[...], 0.0)
    dinv = _dinv_of(hist_ref)
    h0_ref[...] = h
    g_ref[...] = h * dinv


def _layer_body(beta, acc_ref, g_ref, h0_ref, hist_ref, w_ref, b_ref,
                h_ref, gout_ref):
    dinv = _dinv_of(hist_ref)
    hi = dinv * (acc_ref[0] + acc_ref[1] + g_ref[...])
    sup = (1.0 - ALPHA) * hi + ALPHA * h0_ref[...]
    out = (beta * jnp.dot(sup, w_ref[...], preferred_element_type=jnp.float32)
           + (1.0 - beta) * sup + b_ref[...])
    h = jnp.maximum(out, 0.0)
    h_ref[...] = h
    gout_ref[...] = h * dinv


def _out_body(h_ref, w_ref, b_ref, o_ref):
    o_ref[...] = (
        jnp.dot(h_ref[...], w_ref[...], preferred_element_type=jnp.float32)
        + b_ref[...])


_row_spec = pl.BlockSpec((_BLK, H), lambda i: (i, 0))
_mat_spec = pl.BlockSpec((H, H), lambda i: (0, 0))
_vec_spec = pl.BlockSpec((1, H), lambda i: (0, 0))
_hist_spec = pl.BlockSpec((NC, _BLK, 16), lambda i: (0, i, 0))
_acc_spec = pl.BlockSpec((NC, _BLK, H), lambda i: (0, i, 0))
_rows_out = jax.ShapeDtypeStruct((N, H), jnp.float32)


def _pre_call(x, w, b, hist):
    return pl.pallas_call(
        _pre_body,
        grid=(_GRID,),
        in_specs=[_row_spec, _mat_spec, _vec_spec, _hist_spec],
        out_specs=[_row_spec, _row_spec],
        out_shape=[_rows_out, _rows_out],
    )(x, w, b, hist)


def _layer_call(beta, acc, g, h0, hist, w, b):
    return pl.pallas_call(
        functools.partial(_layer_body, beta),
        grid=(_GRID,),
        in_specs=[_acc_spec, _row_spec, _row_spec, _hist_spec, _mat_spec,
                  _vec_spec],
        out_specs=[_row_spec, _row_spec],
        out_shape=[_rows_out, _rows_out],
    )(acc, g, h0, hist, w, b)


def _out_call(h, w, b):
    return pl.pallas_call(
        _out_body,
        grid=(_GRID,),
        in_specs=[_row_spec, _mat_spec, _vec_spec],
        out_specs=_row_spec,
        out_shape=_rows_out,
    )(h, w, b)


def kernel(x, edge_index, W_in, b_in, Wl, bl, W_out, b_out):
    src = edge_index[0]
    dst = edge_index[1]
    pad = E_PAD - E
    pad_hist = jnp.full((pad,), N, dtype=jnp.int32)
    pad_zero = jnp.zeros((pad,), dtype=jnp.int32)
    src_h = jnp.concatenate([src, pad_hist])
    src_g = jnp.concatenate([src, pad_zero])
    dst_p = jnp.concatenate([dst, pad_hist])

    b_in2 = b_in.reshape(1, H)
    w_out_p = jnp.pad(W_out, ((0, 0), (0, H - C)))
    b_out_p = jnp.pad(b_out, (0, H - C)).reshape(1, H)

    hist = _hist_kernel(src_h)
    h, g = _pre_call(x, W_in, b_in2, hist)
    h0 = h
    for i in range(L):
        beta = math.log(LAMDA / (i + 1) + 1.0)
        acc = _spmm_kernel(g, src_g, dst_p)
        h, g = _layer_call(beta, acc, g, h0, hist, Wl[i],
                           bl[i].reshape(1, H))
    out = _out_call(h, w_out_p, b_out_p)
    return out[:, :C]


# speedup vs baseline: 6.7847x; 1.0005x over previous
import functools
import math

import jax
import jax.numpy as jnp
from jax import lax
from jax.experimental import pallas as pl
from jax.experimental.pallas import tpu as pltpu
from jax.experimental.pallas import tpu_sc as plsc

N = 10000
E = 320000
D = 128
H = 128
C = 40
L = 8
ALPHA = 0.1
LAMDA = 0.5

NC = 2
NS = 16
NT = NC * NS
CH = 128
NROW = 640
N_PAD = NS * NROW
E_PW = ((E // NT + CH - 1) // CH) * CH
N_CHUNK = E_PW // CH
E_PAD = E_PW * NT

_mesh = plsc.VectorSubcoreMesh(core_axis_name="c", subcore_axis_name="s")


@functools.partial(
    pl.kernel,
    out_type=jax.ShapeDtypeStruct((NC, N_PAD, 16), jnp.float32),
    mesh=_mesh,
    scratch_types=[
        pltpu.VMEM((CH,), jnp.int32),
        pltpu.VMEM((CH, 16), jnp.float32),
        pltpu.VMEM((CH, 16), jnp.float32),
        pltpu.VMEM_SHARED((N_PAD, 16), jnp.float32),
    ],
)
def _hist_kernel(src_hbm, hist_out, idx_v, obuf, zbuf, hist_sh):
    c = lax.axis_index("c")
    s = lax.axis_index("s")
    t = c * NS + s

    def fill(i, _):
        zbuf[i, :] = jnp.zeros((16,), jnp.float32)
        obuf[i, :] = jnp.ones((16,), jnp.float32)
        return 0

    lax.fori_loop(0, CH, fill, 0)
    for r in range(NROW // CH):
        pltpu.sync_copy(zbuf, hist_sh.at[pl.ds(s * NROW + r * CH, CH)])
    plsc.subcore_barrier()

    def chunk(k, _):
        base = t * E_PW + k * CH
        pltpu.sync_copy(src_hbm.at[pl.ds(base, CH)], idx_v)
        pltpu.sync_copy(obuf, hist_sh.at[idx_v], add=True)
        return 0

    lax.fori_loop(0, N_CHUNK, chunk, 0)
    plsc.subcore_barrier()
    pltpu.sync_copy(hist_sh.at[pl.ds(s * NROW, NROW)],
                    hist_out.at[c, pl.ds(s * NROW, NROW)])


@functools.partial(
    pl.kernel,
    out_type=jax.ShapeDtypeStruct((NC, N_PAD, H), jnp.float32),
    mesh=_mesh,
    scratch_types=[
        pltpu.VMEM((CH,), jnp.int32),
        pltpu.VMEM((CH,), jnp.int32),
        pltpu.VMEM((CH, H), jnp.float32),
        pltpu.VMEM((CH, H), jnp.float32),
        pltpu.VMEM_SHARED((N_PAD, H), jnp.float32),
        pltpu.SemaphoreType.DMA,
    ],
)
def _spmm_kernel(g_hbm, src_hbm, dst_hbm, acc_out,
                 idx_s, idx_d, rows, zbuf, acc_sh, sem):
    c = lax.axis_index("c")
    s = lax.axis_index("s")
    t = c * NS + s

    def fill(i, _):
        for j in range(H // 16):
            zbuf[i, pl.ds(j * 16, 16)] = jnp.zeros((16,), jnp.float32)
        return 0

    lax.fori_loop(0, CH, fill, 0)
    for r in range(NROW // CH):
        pltpu.sync_copy(zbuf, acc_sh.at[pl.ds(s * NROW + r * CH, CH)])
    plsc.subcore_barrier()

    def chunk(k, _):
        base = t * E_PW + k * CH
        pltpu.sync_copy(src_hbm.at[pl.ds(base, CH)], idx_s)
        pltpu.sync_copy(dst_hbm.at[pl.ds(base, CH)], idx_d)
        pltpu.async_copy(g_hbm.at[idx_s], rows, sem).wait()
        pltpu.sync_copy(rows, acc_sh.at[idx_d], add=True)
        return 0

    lax.fori_loop(0, N_CHUNK, chunk, 0)
    plsc.subcore_barrier()
    pltpu.sync_copy(acc_sh.at[pl.ds(s * NROW, NROW)],
                    acc_out.at[c, pl.ds(s * NROW, NROW)])


_BLK = 2000
_GRID = N // _BLK


def _dinv_of(hist_ref):
    deg = hist_ref[0, :, 0:1] + hist_ref[1, :, 0:1] + 1.0
    return lax.rsqrt(deg)


def _pre_body(x_ref, w_ref, b_ref, hist_ref, h0_ref, g_ref):
    h = jnp.maximum(
        jnp.dot(x_ref[...], w_ref[...], preferred_element_type=jnp.float32)
        + b_ref[...], 0.0)
    dinv = _dinv_of(hist_ref)
    h0_ref[...] = h
    g_ref[...] = h * dinv


def _layer_body(beta, acc_ref, g_ref, h0_ref, hist_ref, w_ref, b_ref,
                h_ref, gout_ref):
    dinv = _dinv_of(hist_ref)
    hi = dinv * (acc_ref[0] + acc_ref[1] + g_ref[...])
    sup = (1.0 - ALPHA) * hi + ALPHA * h0_ref[...]
    out = (beta * jnp.dot(sup, w_ref[...], preferred_element_type=jnp.float32)
           + (1.0 - beta) * sup + b_ref[...])
    h = jnp.maximum(out, 0.0)
    h_ref[...] = h
    gout_ref[...] = h * dinv


def _out_body(h_ref, w_ref, b_ref, o_ref):
    o_ref[...] = (
        jnp.dot(h_ref[...], w_ref[...], preferred_element_type=jnp.float32)
        + b_ref[...])


_row_spec = pl.BlockSpec((_BLK, H), lambda i: (i, 0))
_mat_spec = pl.BlockSpec((H, H), lambda i: (0, 0))
_vec_spec = pl.BlockSpec((1, H), lambda i: (0, 0))
_hist_spec = pl.BlockSpec((NC, _BLK, 16), lambda i: (0, i, 0))
_acc_spec = pl.BlockSpec((NC, _BLK, H), lambda i: (0, i, 0))
_rows_out = jax.ShapeDtypeStruct((N, H), jnp.float32)


def _pre_call(x, w, b, hist):
    return pl.pallas_call(
        _pre_body,
        grid=(_GRID,),
        in_specs=[_row_spec, _mat_spec, _vec_spec, _hist_spec],
        out_specs=[_row_spec, _row_spec],
        out_shape=[_rows_out, _rows_out],
    )(x, w, b, hist)


def _layer_call(beta, acc, g, h0, hist, w, b):
    return pl.pallas_call(
        functools.partial(_layer_body, beta),
        grid=(_GRID,),
        in_specs=[_acc_spec, _row_spec, _row_spec, _hist_spec, _mat_spec,
                  _vec_spec],
        out_specs=[_row_spec, _row_spec],
        out_shape=[_rows_out, _rows_out],
    )(acc, g, h0, hist, w, b)


def _out_call(h, w, b):
    return pl.pallas_call(
        _out_body,
        grid=(_GRID,),
        in_specs=[_row_spec, _mat_spec, _vec_spec],
        out_specs=_row_spec,
        out_shape=_rows_out,
    )(h, w, b)


def kernel(x, edge_index, W_in, b_in, Wl, bl, W_out, b_out):
    src = edge_index[0]
    dst = edge_index[1]
    pad = E_PAD - E
    pad_hist = jnp.full((pad,), N, dtype=jnp.int32)
    pad_zero = jnp.zeros((pad,), dtype=jnp.int32)
    src_h = jnp.concatenate([src, pad_hist])
    src_g = jnp.concatenate([src, pad_zero])
    dst_p = jnp.concatenate([dst, pad_hist])

    b_in2 = b_in.reshape(1, H)
    w_out_p = jnp.pad(W_out, ((0, 0), (0, H - C)))
    b_out_p = jnp.pad(b_out, (0, H - C)).reshape(1, H)

    hist = _hist_kernel(src_h)
    h, g = _pre_call(x, W_in, b_in2, hist)
    h0 = h
    for i in range(L):
        beta = math.log(LAMDA / (i + 1) + 1.0)
        acc = _spmm_kernel(g, src_g, dst_p)
        h, g = _layer_call(beta, acc, g, h0, hist, Wl[i],
                           bl[i].reshape(1, H))
    out = _out_call(h, w_out_p, b_out_p)
    return out[:, :C]
